# Initial kernel scaffold; baseline (speedup 1.0000x reference)
import jax
import jax.numpy as jnp
from jax.experimental import pallas as pl


def kernel(x, vectors, w, W_out, b_out):
    raise NotImplementedError("write your pallas kernel here")


# speedup vs baseline: 26.5960x; 26.5960x over previous
import functools

import jax
import jax.numpy as jnp
from jax import lax
from jax.experimental import pallas as pl
from jax.experimental.pallas import tpu as pltpu
from jax.experimental.pallas import tpu_sc as plsc

LANES = 16
NC = 2
NS = 16
NW = NC * NS


def _sc_pool(B, L, V, D, NCHUNK, LC):
  b_per_w = B // NW
  n_d = D // LANES
  LCP = ((LC + LANES - 1) // LANES) * LANES
  n_g = LCP // LANES
  mesh = plsc.VectorSubcoreMesh(core_axis_name="c", subcore_axis_name="s")

  def body(x_hbm, w_hbm, tab_hbm, out_hbm, idx_v, rows_v, wv_v, stage_v, gsem):
    wid = lax.axis_index("s") * NC + lax.axis_index("c")
    base = wid * b_per_w
    pltpu.sync_copy(x_hbm.at[pl.ds(base, b_per_w)], idx_v)

    zvec = jnp.zeros((LANES,), jnp.float32)
    if LCP != LC:
      pad_lo = (LC // LANES) * LANES
      for j in range(NCHUNK):
        wv_v[j, pl.ds(pad_lo, LANES)] = jnp.full((LANES,), -1e30, jnp.float32)
        for t in range(LC, LCP):
          for k in range(n_d):
            rows_v[j, t, pl.ds(k * LANES, LANES)] = zvec

    def row_body(r, _):
      for j in range(NCHUNK):
        pltpu.async_copy(tab_hbm.at[idx_v.at[r, j]], rows_v.at[j, pl.ds(0, LC)], gsem)
        pltpu.async_copy(w_hbm.at[idx_v.at[r, j]], wv_v.at[j, pl.ds(0, LC)], gsem)
      for j in range(NCHUNK):
        pltpu.make_async_copy(tab_hbm.at[idx_v.at[r, j]], rows_v.at[j, pl.ds(0, LC)], gsem).wait()
        pltpu.make_async_copy(w_hbm.at[idx_v.at[r, j]], wv_v.at[j, pl.ds(0, LC)], gsem).wait()

      init = tuple(jnp.zeros((LANES,), jnp.float32) for _ in range(n_d + 1))
      carry = init
      for j in range(NCHUNK):

        def grp_body(g, c, j=j):
          accs, swx = c[:-1], c[-1]
          wraw16 = wv_v[j, pl.ds(g * LANES, LANES)]
          wx16 = 1.0 / (1.0 + jnp.exp(-wraw16))
          accs = list(accs)
          for jj in range(LANES):
            t = g * LANES + jj
            wb = jnp.full((LANES,), wx16[jj], jnp.float32)
            for k in range(n_d):
              accs[k] = accs[k] + wb * rows_v[j, t, pl.ds(k * LANES, LANES)]
          return tuple(accs) + (swx + wx16,)

        carry = lax.fori_loop(0, n_g, grp_body, carry)

      parts = [carry[-1][i] for i in range(LANES)]
      while len(parts) > 1:
        parts = [a + b for a, b in zip(parts[::2], parts[1::2])]
      denom = jnp.full((LANES,), parts[0] + 1e-16, jnp.float32)
      for k in range(n_d):
        stage_v[r, pl.ds(k * LANES, LANES)] = carry[k] / denom
      return 0

    lax.fori_loop(0, b_per_w, row_body, 0)
    pltpu.sync_copy(stage_v, out_hbm.at[pl.ds(base, b_per_w)])

  return pl.kernel(
      body,
      out_type=jax.ShapeDtypeStruct((B, D), jnp.float32),
      mesh=mesh,
      scratch_types=[
          pltpu.VMEM((b_per_w, NCHUNK, LC), jnp.int32),
          pltpu.VMEM((NCHUNK, LCP, D), jnp.float32),
          pltpu.VMEM((NCHUNK, LCP), jnp.float32),
          pltpu.VMEM((b_per_w, D), jnp.float32),
          pltpu.SemaphoreType.DMA,
      ],
  )


def _tc_head(B, D, OUT):
  BM = 512

  def body(a_ref, w_ref, b_ref, o_ref):
    o_ref[...] = (
        lax.dot_general(
            a_ref[...], w_ref[...], (((1,), (1,)), ((), ())),
            preferred_element_type=jnp.float32,
        )
        + b_ref[...]
    )

  return pl.pallas_call(
      body,
      grid=(B // BM,),
      in_specs=[
          pl.BlockSpec((BM, D), lambda i: (i, 0)),
          pl.BlockSpec((OUT, D), lambda i: (0, 0)),
          pl.BlockSpec((1, OUT), lambda i: (0, 0)),
      ],
      out_specs=pl.BlockSpec((BM, OUT), lambda i: (i, 0)),
      out_shape=jax.ShapeDtypeStruct((B, OUT), jnp.float32),
  )


def kernel(x, vectors, w, W_out, b_out):
  B, L = x.shape
  V, D = vectors.shape
  OUT = W_out.shape[0]
  NCHUNK = 2
  LC = L // NCHUNK
  x3 = x.reshape(B, NCHUNK, LC)
  wa = _sc_pool(B, L, V, D, NCHUNK, LC)(x3, w, vectors)
  logits = _tc_head(B, D, OUT)(wa, W_out, b_out.reshape(1, OUT))
  return (logits, wa)


# speedup vs baseline: 41.9758x; 1.5783x over previous
import functools

import jax
import jax.numpy as jnp
from jax import lax
from jax.experimental import pallas as pl
from jax.experimental.pallas import tpu as pltpu
from jax.experimental.pallas import tpu_sc as plsc

LANES = 16
NC = 2
NS = 16
NW = NC * NS


def _sc_pool(B, L, V, D, NCHUNK, LC):
  b_per_w = B // NW
  n_d = D // LANES
  LCP = ((LC + LANES - 1) // LANES) * LANES
  n_g = LCP // LANES
  mesh = plsc.VectorSubcoreMesh(core_axis_name="c", subcore_axis_name="s")

  def body(x_hbm, w_hbm, tab_hbm, out_hbm, idx_v, rows_v, wv_v, stage_v, sems):
    wid = lax.axis_index("s") * NC + lax.axis_index("c")
    base = wid * b_per_w
    pltpu.sync_copy(x_hbm.at[pl.ds(base, b_per_w)], idx_v)

    zvec = jnp.zeros((LANES,), jnp.float32)
    if LCP != LC:
      pad_lo = (LC // LANES) * LANES
      for p in range(2):
        for j in range(NCHUNK):
          wv_v[p, j, pl.ds(pad_lo, LANES)] = jnp.full((LANES,), -1e30, jnp.float32)
          for t in range(LC, LCP):
            for k in range(n_d):
              rows_v[p, j, t, pl.ds(k * LANES, LANES)] = zvec

    def fire(r, p):
      for j in range(NCHUNK):
        pltpu.async_copy(tab_hbm.at[idx_v.at[r, j]], rows_v.at[p, j, pl.ds(0, LC)], sems.at[p])
        pltpu.async_copy(w_hbm.at[idx_v.at[r, j]], wv_v.at[p, j, pl.ds(0, LC)], sems.at[p])

    def drain(r, p):
      for j in range(NCHUNK):
        pltpu.make_async_copy(tab_hbm.at[idx_v.at[r, j]], rows_v.at[p, j, pl.ds(0, LC)], sems.at[p]).wait()
        pltpu.make_async_copy(w_hbm.at[idx_v.at[r, j]], wv_v.at[p, j, pl.ds(0, LC)], sems.at[p]).wait()

    def compute(r, p):
      carry = tuple(jnp.zeros((LANES,), jnp.float32) for _ in range(n_d + 1))
      for j in range(NCHUNK):

        def grp_body(g, c, j=j):
          accs, swx = c[:-1], c[-1]
          wraw16 = wv_v[p, j, pl.ds(g * LANES, LANES)]
          wx16 = 1.0 / (1.0 + jnp.exp(-wraw16))
          accs = list(accs)
          for jj in range(LANES):
            t = g * LANES + jj
            wb = jnp.full((LANES,), wx16[jj], jnp.float32)
            for k in range(n_d):
              accs[k] = accs[k] + wb * rows_v[p, j, t, pl.ds(k * LANES, LANES)]
          return tuple(accs) + (swx + wx16,)

        carry = lax.fori_loop(0, n_g, grp_body, carry)

      parts = [carry[-1][i] for i in range(LANES)]
      while len(parts) > 1:
        parts = [a + b for a, b in zip(parts[::2], parts[1::2])]
      denom = jnp.full((LANES,), parts[0] + 1e-16, jnp.float32)
      for k in range(n_d):
        stage_v[r, pl.ds(k * LANES, LANES)] = carry[k] / denom

    n_pair = b_per_w // 2
    fire(0, 0)

    def pair_body(g, _):
      r0 = 2 * g
      fire(r0 + 1, 1)
      drain(r0, 0)
      compute(r0, 0)

      @pl.when(g < n_pair - 1)
      def _():
        fire(r0 + 2, 0)

      drain(r0 + 1, 1)
      compute(r0 + 1, 1)
      return 0

    lax.fori_loop(0, n_pair, pair_body, 0)
    pltpu.sync_copy(stage_v, out_hbm.at[pl.ds(base, b_per_w)])

  return pl.kernel(
      body,
      out_type=jax.ShapeDtypeStruct((B, D), jnp.float32),
      mesh=mesh,
      scratch_types=[
          pltpu.VMEM((b_per_w, NCHUNK, LC), jnp.int32),
          pltpu.VMEM((2, NCHUNK, LCP, D), jnp.float32),
          pltpu.VMEM((2, NCHUNK, LCP), jnp.float32),
          pltpu.VMEM((b_per_w, D), jnp.float32),
          pltpu.SemaphoreType.DMA((2,)),
      ],
  )


def _tc_head(B, D, OUT):
  BM = 512

  def body(a_ref, w_ref, b_ref, o_ref):
    o_ref[...] = (
        lax.dot_general(
            a_ref[...], w_ref[...], (((1,), (1,)), ((), ())),
            preferred_element_type=jnp.float32,
        )
        + b_ref[...]
    )

  return pl.pallas_call(
      body,
      grid=(B // BM,),
      in_specs=[
          pl.BlockSpec((BM, D), lambda i: (i, 0)),
          pl.BlockSpec((OUT, D), lambda i: (0, 0)),
          pl.BlockSpec((1, OUT), lambda i: (0, 0)),
      ],
      out_specs=pl.BlockSpec((BM, OUT), lambda i: (i, 0)),
      out_shape=jax.ShapeDtypeStruct((B, OUT), jnp.float32),
  )


def kernel(x, vectors, w, W_out, b_out):
  B, L = x.shape
  V, D = vectors.shape
  OUT = W_out.shape[0]
  NCHUNK = 2
  LC = L // NCHUNK
  x3 = x.reshape(B, NCHUNK, LC)
  wa = _sc_pool(B, L, V, D, NCHUNK, LC)(x3, w, vectors)
  logits = _tc_head(B, D, OUT)(wa, W_out, b_out.reshape(1, OUT))
  return (logits, wa)
